# baseline (speedup 1.0000x reference)
import functools
import math

import jax
import jax.numpy as jnp
from jax import lax
from jax.experimental import pallas as pl
from jax.experimental.pallas import tpu as pltpu
from jax.experimental.pallas import tpu_sc as plsc

_MODEL_DIM = 64
_SCALE = math.sqrt(_MODEL_DIM)

_NC = 2
_NS = 16
_NW = _NC * _NS
_CHUNK = 128
_LANES = 16


@functools.lru_cache(maxsize=None)
def _make_lookup(B, D):
    assert B % (_NW * _CHUNK) == 0
    n_chunks_total = B // _CHUNK
    chunks_per_w = n_chunks_total // _NW
    mesh = plsc.VectorSubcoreMesh(core_axis_name="c", subcore_axis_name="s")

    @functools.partial(
        pl.kernel,
        mesh=mesh,
        out_type=jax.ShapeDtypeStruct((B, D), jnp.float32),
        scratch_types=[
            pltpu.VMEM((chunks_per_w, _CHUNK), jnp.int32),
            pltpu.VMEM((_CHUNK, D), jnp.float32),
            pltpu.SemaphoreType.DMA,
        ],
        compiler_params=pltpu.CompilerParams(use_tc_tiling_on_sc=False),
    )
    def lookup(table_hbm, idx_hbm, out_hbm, idx_v, rows_v, sem):
        wid = lax.axis_index("s") * _NC + lax.axis_index("c")
        row0 = wid * chunks_per_w
        pltpu.sync_copy(idx_hbm.at[pl.ds(row0, chunks_per_w)], idx_v)

        def chunk_body(j, carry):
            pltpu.async_copy(table_hbm.at[idx_v.at[j]], rows_v, sem).wait()

            def scale_row(r, c2):
                for q in range(D // _LANES):
                    sl = pl.ds(q * _LANES, _LANES)
                    rows_v[r, sl] = rows_v[r, sl] * _SCALE
                return c2

            lax.fori_loop(0, _CHUNK, scale_row, 0)
            pltpu.sync_copy(rows_v, out_hbm.at[pl.ds((row0 + j) * _CHUNK, _CHUNK)])
            return carry

        lax.fori_loop(0, chunks_per_w, chunk_body, 0)

    return lookup


def kernel(input_ids, embeddings):
    batch, hist = input_ids.shape
    _, d = embeddings.shape
    b = batch * hist
    idx2d = input_ids.astype(jnp.int32).reshape(b // _CHUNK, _CHUNK)
    out = _make_lookup(b, d)(embeddings, idx2d)
    return out.reshape(batch, hist, d)


# speedup vs baseline: 1.2063x; 1.2063x over previous
import functools
import math

import jax
import jax.numpy as jnp
from jax import lax
from jax.experimental import pallas as pl
from jax.experimental.pallas import tpu as pltpu
from jax.experimental.pallas import tpu_sc as plsc

_SCALE = math.sqrt(64.0)

_NC = 2
_NS = 16
_NW = _NC * _NS
_CHUNK = 128

_T_BLK = 1024


def _prescale_body(x1_ref, x2_ref, o_ref):
    o_ref[...] = jnp.concatenate(
        [x1_ref[...].T, x2_ref[...].T], axis=1) * _SCALE


@functools.lru_cache(maxsize=None)
def _make_prescale_transpose(v, d):
    grid = (v + _T_BLK - 1) // _T_BLK
    half = _T_BLK // 2
    return pl.pallas_call(
        _prescale_body,
        grid=(grid,),
        in_specs=[
            pl.BlockSpec((d, half), lambda k: (0, 2 * k)),
            pl.BlockSpec((d, half), lambda k: (0, 2 * k + 1)),
        ],
        out_specs=pl.BlockSpec((half, 2 * d), lambda k: (k, 0)),
        out_shape=jax.ShapeDtypeStruct((grid * half, 2 * d), jnp.float32),
    )


def _view_row(r):
    a = jnp.bitwise_and(r, ~(_T_BLK - 1))
    h = jnp.bitwise_and(jnp.right_shift(r, 9), 1)
    p = jnp.bitwise_and(r, _T_BLK // 2 - 1)
    return a + 2 * p + h


def _out_body(x_ref, o_ref):
    o_ref[...] = x_ref[...].T


@functools.lru_cache(maxsize=None)
def _make_out_transpose(batch, hd):
    return pl.pallas_call(
        _out_body,
        grid=(batch // 128,),
        in_specs=[pl.BlockSpec((128, hd), lambda j: (j, 0))],
        out_specs=pl.BlockSpec((hd, 128), lambda j: (0, j)),
        out_shape=jax.ShapeDtypeStruct((hd, batch), jnp.float32),
    )


@functools.lru_cache(maxsize=None)
def _make_gather(b, v, d):
    assert b % (_NW * _CHUNK) == 0
    chunks_per_w = b // _CHUNK // _NW
    assert chunks_per_w % 2 == 0
    mesh = plsc.VectorSubcoreMesh(core_axis_name="c", subcore_axis_name="s")

    @functools.partial(
        pl.kernel,
        mesh=mesh,
        out_type=jax.ShapeDtypeStruct((b // _CHUNK, _CHUNK, d), jnp.float32),
        scratch_types=[
            pltpu.VMEM((chunks_per_w, _CHUNK), jnp.int32),
            pltpu.VMEM((_CHUNK, d), jnp.float32),
            pltpu.VMEM((_CHUNK, d), jnp.float32),
            pltpu.SemaphoreType.DMA,
            pltpu.SemaphoreType.DMA,
        ],
        compiler_params=pltpu.CompilerParams(use_tc_tiling_on_sc=False),
    )
    def lookup(table_hbm, idx_hbm, out_hbm, idx_v, buf_a, buf_b, sem_a, sem_b):
        wid = lax.axis_index("s") * _NC + lax.axis_index("c")
        row0 = wid * chunks_per_w
        pltpu.sync_copy(idx_hbm.at[pl.ds(row0, chunks_per_w)], idx_v)

        pltpu.async_copy(table_hbm.at[idx_v.at[0]], buf_a, sem_a)
        pltpu.async_copy(table_hbm.at[idx_v.at[1]], buf_b, sem_b)

        def group(g, carry):
            j = 2 * g
            pltpu.make_async_copy(table_hbm.at[idx_v.at[j]], buf_a, sem_a).wait()
            pltpu.sync_copy(buf_a, out_hbm.at[row0 + j])

            @pl.when(j + 2 < chunks_per_w)
            def _():
                pltpu.async_copy(table_hbm.at[idx_v.at[j + 2]], buf_a, sem_a)

            pltpu.make_async_copy(table_hbm.at[idx_v.at[j + 1]], buf_b, sem_b).wait()
            pltpu.sync_copy(buf_b, out_hbm.at[row0 + j + 1])

            @pl.when(j + 3 < chunks_per_w)
            def _():
                pltpu.async_copy(table_hbm.at[idx_v.at[j + 3]], buf_b, sem_b)

            return carry

        lax.fori_loop(0, chunks_per_w // 2, group, 0)

    return lookup


def kernel(input_ids, embeddings):
    batch, hist = input_ids.shape
    v, d = embeddings.shape
    b = batch * hist

    emb_t = embeddings.T
    tbl2 = _make_prescale_transpose(v, d)(emb_t, emb_t)
    v_view = 2 * tbl2.shape[0]
    tbl = tbl2.reshape(v_view, d)
    idx2d = _view_row(input_ids.astype(jnp.int32)).reshape(b // _CHUNK, _CHUNK)
    out2 = _make_gather(b, v_view, d)(tbl, idx2d)
    out_bhc = out2.reshape(batch, hist * d)
    out_t = _make_out_transpose(batch, hist * d)(out_bhc)
    return out_t.reshape(hist, d, batch).transpose(2, 0, 1)


# speedup vs baseline: 1.8244x; 1.5124x over previous
import functools
import math

import jax
import jax.numpy as jnp
from jax import lax
from jax.experimental import pallas as pl
from jax.experimental.pallas import tpu as pltpu
from jax.experimental.pallas import tpu_sc as plsc

_SCALE = math.sqrt(64.0)

_NC = 2
_NS = 16
_NW = _NC * _NS
_CHUNK = 128

_T_BLK = 8192
_T_HALF_BITS = 12


def _prescale_body(x1_ref, x2_ref, o_ref):
    o_ref[...] = jnp.concatenate(
        [x1_ref[...].T, x2_ref[...].T], axis=1) * _SCALE


@functools.lru_cache(maxsize=None)
def _make_prescale_transpose(v, d):
    grid = (v + _T_BLK - 1) // _T_BLK
    half = _T_BLK // 2
    maxb = (v - 1) // half
    return pl.pallas_call(
        _prescale_body,
        grid=(grid,),
        in_specs=[
            pl.BlockSpec((d, half), lambda k, m=maxb: (0, jnp.minimum(2 * k, m))),
            pl.BlockSpec(
                (d, half), lambda k, m=maxb: (0, jnp.minimum(2 * k + 1, m))),
        ],
        out_specs=pl.BlockSpec((half, 2 * d), lambda k: (k, 0)),
        out_shape=jax.ShapeDtypeStruct((grid * half, 2 * d), jnp.float32),
    )


def _view_row(r):
    a = jnp.bitwise_and(r, ~(_T_BLK - 1))
    h = jnp.bitwise_and(jnp.right_shift(r, _T_HALF_BITS), 1)
    p = jnp.bitwise_and(r, _T_BLK // 2 - 1)
    return a + 2 * p + h


def _out_body(x_ref, o_ref):
    o_ref[...] = x_ref[...].T


@functools.lru_cache(maxsize=None)
def _make_out_transpose(batch, hd):
    return pl.pallas_call(
        _out_body,
        grid=(batch // 128,),
        in_specs=[pl.BlockSpec((128, hd), lambda j: (j, 0))],
        out_specs=pl.BlockSpec((hd, 128), lambda j: (0, j)),
        out_shape=jax.ShapeDtypeStruct((hd, batch), jnp.float32),
    )


@functools.lru_cache(maxsize=None)
def _make_gather(b, v, d):
    assert b % (_NW * _CHUNK) == 0
    chunks_per_w = b // _CHUNK // _NW
    assert chunks_per_w % 2 == 0
    mesh = plsc.VectorSubcoreMesh(core_axis_name="c", subcore_axis_name="s")

    @functools.partial(
        pl.kernel,
        mesh=mesh,
        out_type=jax.ShapeDtypeStruct((b // _CHUNK, _CHUNK, d), jnp.float32),
        scratch_types=[
            pltpu.VMEM((chunks_per_w, _CHUNK), jnp.int32),
            pltpu.VMEM((_CHUNK, d), jnp.float32),
            pltpu.VMEM((_CHUNK, d), jnp.float32),
            pltpu.SemaphoreType.DMA,
            pltpu.SemaphoreType.DMA,
        ],
        compiler_params=pltpu.CompilerParams(use_tc_tiling_on_sc=False),
    )
    def lookup(table_hbm, idx_hbm, out_hbm, idx_v, buf_a, buf_b, sem_a, sem_b):
        wid = lax.axis_index("s") * _NC + lax.axis_index("c")
        row0 = wid * chunks_per_w
        pltpu.sync_copy(idx_hbm.at[pl.ds(row0, chunks_per_w)], idx_v)

        pltpu.async_copy(table_hbm.at[idx_v.at[0]], buf_a, sem_a)
        pltpu.async_copy(table_hbm.at[idx_v.at[1]], buf_b, sem_b)

        def group(g, carry):
            j = 2 * g
            pltpu.make_async_copy(table_hbm.at[idx_v.at[j]], buf_a, sem_a).wait()
            pltpu.sync_copy(buf_a, out_hbm.at[row0 + j])

            @pl.when(j + 2 < chunks_per_w)
            def _():
                pltpu.async_copy(table_hbm.at[idx_v.at[j + 2]], buf_a, sem_a)

            pltpu.make_async_copy(table_hbm.at[idx_v.at[j + 1]], buf_b, sem_b).wait()
            pltpu.sync_copy(buf_b, out_hbm.at[row0 + j + 1])

            @pl.when(j + 3 < chunks_per_w)
            def _():
                pltpu.async_copy(table_hbm.at[idx_v.at[j + 3]], buf_b, sem_b)

            return carry

        lax.fori_loop(0, chunks_per_w // 2, group, 0)

    return lookup


def kernel(input_ids, embeddings):
    batch, hist = input_ids.shape
    v, d = embeddings.shape
    b = batch * hist

    emb_t = embeddings.T
    tbl2 = _make_prescale_transpose(v, d)(emb_t, emb_t)
    v_view = 2 * tbl2.shape[0]
    tbl = tbl2.reshape(v_view, d)
    idx2d = _view_row(input_ids.astype(jnp.int32)).reshape(b // _CHUNK, _CHUNK)
    out2 = _make_gather(b, v_view, d)(tbl, idx2d)
    out_bhc = out2.reshape(batch, hist * d)
    out_t = _make_out_transpose(batch, hist * d)(out_bhc)
    return out_t.reshape(hist, d, batch).transpose(2, 0, 1)


# speedup vs baseline: 1.8968x; 1.0397x over previous
import functools
import math

import jax
import jax.numpy as jnp
from jax import lax
from jax.experimental import pallas as pl
from jax.experimental.pallas import tpu as pltpu
from jax.experimental.pallas import tpu_sc as plsc

_SCALE = math.sqrt(64.0)

_NC = 2
_NS = 16
_NW = _NC * _NS
_CHUNK = 128

_T_BLK = 8192
_T_HALF_BITS = 12


def _prescale_body(x1_ref, x2_ref, o_ref):
    o_ref[...] = jnp.concatenate(
        [x1_ref[...].T, x2_ref[...].T], axis=1) * _SCALE


@functools.lru_cache(maxsize=None)
def _make_prescale_transpose(v, d):
    grid = (v + _T_BLK - 1) // _T_BLK
    half = _T_BLK // 2
    maxb = (v - 1) // half
    return pl.pallas_call(
        _prescale_body,
        grid=(grid,),
        in_specs=[
            pl.BlockSpec((d, half), lambda k, m=maxb: (0, jnp.minimum(2 * k, m))),
            pl.BlockSpec(
                (d, half), lambda k, m=maxb: (0, jnp.minimum(2 * k + 1, m))),
        ],
        out_specs=pl.BlockSpec((half, 2 * d), lambda k: (k, 0)),
        out_shape=jax.ShapeDtypeStruct((grid * half, 2 * d), jnp.float32),
    )


def _view_row(r):
    a = jnp.bitwise_and(r, ~(_T_BLK - 1))
    h = jnp.bitwise_and(jnp.right_shift(r, _T_HALF_BITS), 1)
    p = jnp.bitwise_and(r, _T_BLK // 2 - 1)
    return a + 2 * p + h


def _out_body(x_ref, o_ref):
    o_ref[...] = x_ref[...].T


def _out_body_acc(_, x_ref, o_ref):
    o_ref[...] = x_ref[...].T


@functools.lru_cache(maxsize=None)
def _make_out_transpose(batch, hd, col0, total):
    c0 = col0 // 128
    if col0 == 0:
        return pl.pallas_call(
            _out_body,
            grid=(batch // 128,),
            in_specs=[pl.BlockSpec((128, hd), lambda j: (j, 0))],
            out_specs=pl.BlockSpec((hd, 128), lambda j, c=c0: (0, c + j)),
            out_shape=jax.ShapeDtypeStruct((hd, total), jnp.float32),
        )
    return pl.pallas_call(
        _out_body_acc,
        grid=(batch // 128,),
        in_specs=[
            pl.BlockSpec(memory_space=pl.ANY),
            pl.BlockSpec((128, hd), lambda j: (j, 0)),
        ],
        out_specs=pl.BlockSpec((hd, 128), lambda j, c=c0: (0, c + j)),
        out_shape=jax.ShapeDtypeStruct((hd, total), jnp.float32),
        input_output_aliases={0: 0},
    )


@functools.lru_cache(maxsize=None)
def _make_gather(b, v, d):
    assert b % (_NW * _CHUNK) == 0
    chunks_per_w = b // _CHUNK // _NW
    assert chunks_per_w % 2 == 0
    mesh = plsc.VectorSubcoreMesh(core_axis_name="c", subcore_axis_name="s")

    @functools.partial(
        pl.kernel,
        mesh=mesh,
        out_type=jax.ShapeDtypeStruct((b // _CHUNK, _CHUNK, d), jnp.float32),
        scratch_types=[
            pltpu.VMEM((chunks_per_w, _CHUNK), jnp.int32),
            pltpu.VMEM((_CHUNK, d), jnp.float32),
            pltpu.VMEM((_CHUNK, d), jnp.float32),
            pltpu.SemaphoreType.DMA,
            pltpu.SemaphoreType.DMA,
        ],
        compiler_params=pltpu.CompilerParams(use_tc_tiling_on_sc=False),
    )
    def lookup(table_hbm, idx_hbm, out_hbm, idx_v, buf_a, buf_b, sem_a, sem_b):
        wid = lax.axis_index("s") * _NC + lax.axis_index("c")
        row0 = wid * chunks_per_w
        pltpu.sync_copy(idx_hbm.at[pl.ds(row0, chunks_per_w)], idx_v)

        pltpu.async_copy(table_hbm.at[idx_v.at[0]], buf_a, sem_a)
        pltpu.async_copy(table_hbm.at[idx_v.at[1]], buf_b, sem_b)

        def group(g, carry):
            j = 2 * g
            pltpu.make_async_copy(table_hbm.at[idx_v.at[j]], buf_a, sem_a).wait()
            pltpu.sync_copy(buf_a, out_hbm.at[row0 + j])

            @pl.when(j + 2 < chunks_per_w)
            def _():
                pltpu.async_copy(table_hbm.at[idx_v.at[j + 2]], buf_a, sem_a)

            pltpu.make_async_copy(table_hbm.at[idx_v.at[j + 1]], buf_b, sem_b).wait()
            pltpu.sync_copy(buf_b, out_hbm.at[row0 + j + 1])

            @pl.when(j + 3 < chunks_per_w)
            def _():
                pltpu.async_copy(table_hbm.at[idx_v.at[j + 3]], buf_b, sem_b)

            return carry

        lax.fori_loop(0, chunks_per_w // 2, group, 0)

    return lookup


def kernel(input_ids, embeddings):
    batch, hist = input_ids.shape
    v, d = embeddings.shape
    b = batch * hist

    emb_t = embeddings.T
    tbl2 = _make_prescale_transpose(v, d)(emb_t, emb_t)
    v_view = 2 * tbl2.shape[0]
    tbl = tbl2.reshape(v_view, d)
    idx2d = _view_row(input_ids.astype(jnp.int32)).reshape(b // _CHUNK, _CHUNK)

    hd = hist * d
    nrows = idx2d.shape[0]
    gather = _make_gather(b // 2, v_view, d)
    out2a = gather(tbl, idx2d[: nrows // 2])
    out2b = gather(tbl, idx2d[nrows // 2:])
    ha = out2a.reshape(batch // 2, hd)
    hb = out2b.reshape(batch // 2, hd)
    out_t = _make_out_transpose(batch // 2, hd, 0, batch)(ha)
    out_t = _make_out_transpose(batch // 2, hd, batch // 2, batch)(out_t, hb)
    return out_t.reshape(hist, d, batch).transpose(2, 0, 1)


# speedup vs baseline: 1.9284x; 1.0167x over previous
import functools
import math

import jax
import jax.numpy as jnp
from jax import lax
from jax.experimental import pallas as pl
from jax.experimental.pallas import tpu as pltpu
from jax.experimental.pallas import tpu_sc as plsc

_SCALE = math.sqrt(64.0)

_NC = 2
_NS = 16
_NW = _NC * _NS
_CHUNK = 128

_T_BLK = 8192
_T_HALF_BITS = 12


def _prescale_body(x1_ref, x2_ref, o_ref):
    o_ref[...] = jnp.concatenate(
        [x1_ref[...].T, x2_ref[...].T], axis=1) * _SCALE


@functools.lru_cache(maxsize=None)
def _make_prescale_transpose(v, d):
    grid = (v + _T_BLK - 1) // _T_BLK
    half = _T_BLK // 2
    maxb = (v - 1) // half
    return pl.pallas_call(
        _prescale_body,
        grid=(grid,),
        in_specs=[
            pl.BlockSpec((d, half), lambda k, m=maxb: (0, jnp.minimum(2 * k, m))),
            pl.BlockSpec(
                (d, half), lambda k, m=maxb: (0, jnp.minimum(2 * k + 1, m))),
        ],
        out_specs=pl.BlockSpec((half, 2 * d), lambda k: (k, 0)),
        out_shape=jax.ShapeDtypeStruct((grid * half, 2 * d), jnp.float32),
    )


def _view_row(r):
    a = jnp.bitwise_and(r, ~(_T_BLK - 1))
    h = jnp.bitwise_and(jnp.right_shift(r, _T_HALF_BITS), 1)
    p = jnp.bitwise_and(r, _T_BLK // 2 - 1)
    return a + 2 * p + h


def _out_body(x_ref, o_ref):
    o_ref[...] = x_ref[...].T


def _out_body_acc(_, x_ref, o_ref):
    o_ref[...] = x_ref[...].T


@functools.lru_cache(maxsize=None)
def _make_out_transpose(batch, hd, col0, total):
    c0 = col0 // 128
    if col0 == 0:
        return pl.pallas_call(
            _out_body,
            grid=(batch // 128,),
            in_specs=[pl.BlockSpec((128, hd), lambda j: (j, 0))],
            out_specs=pl.BlockSpec((hd, 128), lambda j, c=c0: (0, c + j)),
            out_shape=jax.ShapeDtypeStruct((hd, total), jnp.float32),
        )
    return pl.pallas_call(
        _out_body_acc,
        grid=(batch // 128,),
        in_specs=[
            pl.BlockSpec(memory_space=pl.ANY),
            pl.BlockSpec((128, hd), lambda j: (j, 0)),
        ],
        out_specs=pl.BlockSpec((hd, 128), lambda j, c=c0: (0, c + j)),
        out_shape=jax.ShapeDtypeStruct((hd, total), jnp.float32),
        input_output_aliases={0: 0},
    )


@functools.lru_cache(maxsize=None)
def _make_gather(b, v, d):
    assert b % (_NW * _CHUNK) == 0
    chunks_per_w = b // _CHUNK // _NW
    assert chunks_per_w % 2 == 0
    mesh = plsc.VectorSubcoreMesh(core_axis_name="c", subcore_axis_name="s")

    @functools.partial(
        pl.kernel,
        mesh=mesh,
        out_type=jax.ShapeDtypeStruct((b // _CHUNK, _CHUNK, d), jnp.float32),
        scratch_types=[
            pltpu.VMEM((chunks_per_w, _CHUNK), jnp.int32),
            pltpu.VMEM((_CHUNK, d), jnp.float32),
            pltpu.VMEM((_CHUNK, d), jnp.float32),
            pltpu.SemaphoreType.DMA,
            pltpu.SemaphoreType.DMA,
        ],
        compiler_params=pltpu.CompilerParams(use_tc_tiling_on_sc=False),
    )
    def lookup(table_hbm, idx_hbm, out_hbm, idx_v, buf_a, buf_b, sem_a, sem_b):
        wid = lax.axis_index("s") * _NC + lax.axis_index("c")
        row0 = wid * chunks_per_w
        pltpu.sync_copy(idx_hbm.at[pl.ds(row0, chunks_per_w)], idx_v)

        pltpu.async_copy(table_hbm.at[idx_v.at[0]], buf_a, sem_a)
        pltpu.async_copy(table_hbm.at[idx_v.at[1]], buf_b, sem_b)

        def group(g, carry):
            j = 2 * g
            pltpu.make_async_copy(table_hbm.at[idx_v.at[j]], buf_a, sem_a).wait()
            pltpu.sync_copy(buf_a, out_hbm.at[row0 + j])

            @pl.when(j + 2 < chunks_per_w)
            def _():
                pltpu.async_copy(table_hbm.at[idx_v.at[j + 2]], buf_a, sem_a)

            pltpu.make_async_copy(table_hbm.at[idx_v.at[j + 1]], buf_b, sem_b).wait()
            pltpu.sync_copy(buf_b, out_hbm.at[row0 + j + 1])

            @pl.when(j + 3 < chunks_per_w)
            def _():
                pltpu.async_copy(table_hbm.at[idx_v.at[j + 3]], buf_b, sem_b)

            return carry

        lax.fori_loop(0, chunks_per_w // 2, group, 0)

    return lookup


def kernel(input_ids, embeddings):
    batch, hist = input_ids.shape
    v, d = embeddings.shape
    b = batch * hist

    emb_t = embeddings.T
    tbl2 = _make_prescale_transpose(v, d)(emb_t, emb_t)
    v_view = 2 * tbl2.shape[0]
    tbl = tbl2.reshape(v_view, d)
    idx2d = _view_row(input_ids.astype(jnp.int32)).reshape(b // _CHUNK, _CHUNK)

    rounds = 4
    hd = hist * d
    qb = batch // rounds
    qrows = idx2d.shape[0] // rounds
    gather = _make_gather(b // rounds, v_view, d)
    parts = [
        gather(tbl, idx2d[i * qrows:(i + 1) * qrows]).reshape(qb, hd)
        for i in range(rounds)
    ]
    out_t = _make_out_transpose(qb, hd, 0, batch)(parts[0])
    for i in range(1, rounds):
        out_t = _make_out_transpose(qb, hd, i * qb, batch)(out_t, parts[i])
    return out_t.reshape(hist, d, batch).transpose(2, 0, 1)
